# baseline (speedup 1.0000x reference)
import jax
import jax.numpy as jnp
from jax.experimental import pallas as pl
from jax.experimental.pallas import tpu as pltpu

HP = 256
G4 = 4 * HP
COUT = 128
N_LAYERS = 4
N_CLASSES = 10
OFF = 64
XW = 256
N_CHUNKS = 2


def _shift_gate_cols(w, off):
    parts = []
    for g in range(4):
        slab = w[..., g * HP:(g + 1) * HP - off]
        pad = [(0, 0)] * (w.ndim - 1) + [(off, 0)]
        parts.append(jnp.pad(slab, pad))
    return jnp.concatenate(parts, axis=-1)


def _shift_rows(w, off):
    pad = [(0, 0)] * (w.ndim - 2) + [(off, 0), (0, 0)]
    return jnp.pad(w[..., :w.shape[-2] - off, :], pad)


def _cell(z, c_prev):
    i_g = jax.nn.sigmoid(z[:, 0 * HP:1 * HP])
    f_g = jax.nn.sigmoid(z[:, 1 * HP:2 * HP])
    g_g = jnp.tanh(z[:, 2 * HP:3 * HP])
    o_g = jax.nn.sigmoid(z[:, 3 * HP:4 * HP])
    c_new = f_g * c_prev + i_g * g_g
    h_new = o_g * jnp.tanh(c_new)
    return h_new, c_new


def _lstm_body(xp_ref, b0_ref, u0c_ref, wcat_ref, br_ref, wfc_ref,
               bfc_ref, out_ref, seq_ref):
    T = xp_ref.shape[0]
    bb = out_ref.shape[0]
    C = N_CHUNKS if bb % (8 * N_CHUNKS) == 0 else 1
    ch = bb // C

    u0c = u0c_ref[...]
    b0 = b0_ref[...]
    cs = [jnp.zeros((ch, HP), jnp.float32)] * C
    hs = [jnp.zeros((ch, HP), jnp.bfloat16)] * C
    for t in range(T):
        for j in range(C):
            lhs = hs[j] + xp_ref[t, j * ch:(j + 1) * ch]
            z = jnp.dot(lhs, u0c, preferred_element_type=jnp.float32) + b0
            h, cs[j] = _cell(z, cs[j])
            hs[j] = h.astype(jnp.bfloat16)
            seq_ref[t, j * ch:(j + 1) * ch] = hs[j]

    n_rest = wcat_ref.shape[0]
    for l in range(n_rest):
        wih = wcat_ref[l, :HP]
        whh = wcat_ref[l, HP:]
        b = br_ref[l]
        cs = [jnp.zeros((ch, HP), jnp.float32)] * C
        hs = [jnp.zeros((ch, HP), jnp.bfloat16)] * C
        for t in range(T):
            for j in range(C):
                z = jnp.dot(seq_ref[t, j * ch:(j + 1) * ch], wih,
                            preferred_element_type=jnp.float32) + b
                if t > 0:
                    z += jnp.dot(hs[j], whh, preferred_element_type=jnp.float32)
                h, cs[j] = _cell(z, cs[j])
                hs[j] = h.astype(jnp.bfloat16)
                if l + 1 < n_rest:
                    seq_ref[t, j * ch:(j + 1) * ch] = hs[j]

    for j in range(C):
        out_ref[j * ch:(j + 1) * ch] = (
            jnp.dot(hs[j], wfc_ref[...],
                    preferred_element_type=jnp.float32) + bfc_ref[...])


def kernel(x, w_ih0, b0, u0, wcat, b_rest, wfc, bfc):
    B, T, F = x.shape
    bf = jnp.bfloat16

    if B % 256 == 0 and B >= 512:
        bb = 256
    elif B % 8 == 0 and B > 8:
        bb = 8
    else:
        bb = B
    grid = (B // bb,)

    xp = jnp.pad(jnp.transpose(x, (1, 0, 2)).astype(bf),
                 ((0, 0), (0, 0), (0, XW - F)))

    u0s = _shift_gate_cols(_shift_rows(u0, OFF), OFF)
    w0s = jnp.pad(_shift_gate_cols(w_ih0, OFF), ((0, HP - F), (0, 0)))
    u0c = (u0s + w0s).astype(bf)
    b0s = _shift_gate_cols(b0, OFF)
    wcats = jnp.concatenate(
        [_shift_rows(wcat[:, :HP], OFF), _shift_rows(wcat[:, HP:], OFF)],
        axis=1)
    wcats = _shift_gate_cols(wcats, OFF).astype(bf)
    brs = _shift_gate_cols(b_rest, OFF)
    wfcs = _shift_rows(wfc, OFF).astype(bf)

    out = pl.pallas_call(
        _lstm_body,
        out_shape=jax.ShapeDtypeStruct((B, COUT), jnp.float32),
        grid=grid,
        in_specs=[
            pl.BlockSpec((T, bb, XW), lambda i: (0, i, 0)),
            pl.BlockSpec((1, G4), lambda i: (0, 0)),
            pl.BlockSpec((HP, G4), lambda i: (0, 0)),
            pl.BlockSpec((N_LAYERS - 1, 2 * HP, G4), lambda i: (0, 0, 0)),
            pl.BlockSpec((N_LAYERS - 1, 1, G4), lambda i: (0, 0, 0)),
            pl.BlockSpec((HP, COUT), lambda i: (0, 0)),
            pl.BlockSpec((1, COUT), lambda i: (0, 0)),
        ],
        out_specs=pl.BlockSpec((bb, COUT), lambda i: (i, 0)),
        scratch_shapes=[pltpu.VMEM((T, bb, HP), jnp.bfloat16)],
        compiler_params=pltpu.CompilerParams(
            dimension_semantics=("parallel",),
            vmem_limit_bytes=64 * 1024 * 1024),
    )(xp, b0s, u0c, wcats, brs, wfcs, bfc)
    return out[:, :N_CLASSES]


# speedup vs baseline: 1.1449x; 1.1449x over previous
import jax
import jax.numpy as jnp
from jax.experimental import pallas as pl
from jax.experimental.pallas import tpu as pltpu

HP = 256
G4 = 4 * HP
COUT = 128
N_LAYERS = 4
N_CLASSES = 10
N_CHUNKS = 2


def _cell4(zi, zf, zg, zo, c_prev):
    i_g = jax.nn.sigmoid(zi)
    f_g = jax.nn.sigmoid(zf)
    g_g = jnp.tanh(zg)
    o_g = jax.nn.sigmoid(zo)
    c_new = f_g * c_prev + i_g * g_g
    h_new = o_g * jnp.tanh(c_new)
    return h_new, c_new


def _gate_z(lhs_x, w_x, lhs_h, w_h, bias, first):
    zs = []
    for g in range(4):
        z = jnp.dot(lhs_x, w_x[:, g * HP:(g + 1) * HP],
                    preferred_element_type=jnp.float32) + bias[:, g * HP:(g + 1) * HP]
        if not first:
            z += jnp.dot(lhs_h, w_h[:, g * HP:(g + 1) * HP],
                         preferred_element_type=jnp.float32)
        zs.append(z)
    return zs


def _lstm_body(xT_ref, w0_ref, b0_ref, u0_ref, wcat_ref, br_ref, wfc_ref,
               bfc_ref, out_ref, seq_ref):
    T = xT_ref.shape[0]
    bb = out_ref.shape[0]
    C = N_CHUNKS if bb % (8 * N_CHUNKS) == 0 else 1
    ch = bb // C

    w0 = w0_ref[...]
    u0 = u0_ref[...]
    b0 = b0_ref[...]
    cs = [jnp.zeros((ch, HP), jnp.float32)] * C
    hs = [jnp.zeros((ch, HP), jnp.bfloat16)] * C
    for t in range(T):
        for j in range(C):
            zs = _gate_z(xT_ref[t, j * ch:(j + 1) * ch], w0, hs[j], u0,
                         b0, t == 0)
            h, cs[j] = _cell4(*zs, cs[j])
            hs[j] = h.astype(jnp.bfloat16)
            seq_ref[t, j * ch:(j + 1) * ch] = hs[j]

    n_rest = wcat_ref.shape[0]
    for l in range(n_rest):
        wih = wcat_ref[l, :HP]
        whh = wcat_ref[l, HP:]
        b = br_ref[l]
        cs = [jnp.zeros((ch, HP), jnp.float32)] * C
        hs = [jnp.zeros((ch, HP), jnp.bfloat16)] * C
        for t in range(T):
            for j in range(C):
                zs = _gate_z(seq_ref[t, j * ch:(j + 1) * ch], wih, hs[j], whh,
                             b, t == 0)
                h, cs[j] = _cell4(*zs, cs[j])
                hs[j] = h.astype(jnp.bfloat16)
                if l + 1 < n_rest:
                    seq_ref[t, j * ch:(j + 1) * ch] = hs[j]

    for j in range(C):
        out_ref[j * ch:(j + 1) * ch] = (
            jnp.dot(hs[j], wfc_ref[...],
                    preferred_element_type=jnp.float32) + bfc_ref[...])


def kernel(x, w_ih0, b0, u0, wcat, b_rest, wfc, bfc):
    B, T, F = x.shape
    bf = jnp.bfloat16

    if B % 256 == 0 and B >= 512:
        bb = 256
    elif B % 8 == 0 and B > 8:
        bb = 8
    else:
        bb = B
    grid = (B // bb,)

    xT = jnp.transpose(x, (1, 0, 2)).astype(bf)

    out = pl.pallas_call(
        _lstm_body,
        out_shape=jax.ShapeDtypeStruct((B, COUT), jnp.float32),
        grid=grid,
        in_specs=[
            pl.BlockSpec((T, bb, F), lambda i: (0, i, 0)),
            pl.BlockSpec((F, G4), lambda i: (0, 0)),
            pl.BlockSpec((1, G4), lambda i: (0, 0)),
            pl.BlockSpec((HP, G4), lambda i: (0, 0)),
            pl.BlockSpec((N_LAYERS - 1, 2 * HP, G4), lambda i: (0, 0, 0)),
            pl.BlockSpec((N_LAYERS - 1, 1, G4), lambda i: (0, 0, 0)),
            pl.BlockSpec((HP, COUT), lambda i: (0, 0)),
            pl.BlockSpec((1, COUT), lambda i: (0, 0)),
        ],
        out_specs=pl.BlockSpec((bb, COUT), lambda i: (i, 0)),
        scratch_shapes=[pltpu.VMEM((T, bb, HP), jnp.bfloat16)],
        compiler_params=pltpu.CompilerParams(
            dimension_semantics=("parallel",),
            vmem_limit_bytes=64 * 1024 * 1024),
    )(xT, w_ih0.astype(bf), b0, u0.astype(bf), wcat.astype(bf),
      b_rest, wfc.astype(bf), bfc)
    return out[:, :N_CLASSES]
